# Initial kernel scaffold; baseline (speedup 1.0000x reference)
import jax
import jax.numpy as jnp
from jax.experimental import pallas as pl


def kernel(x, coeffs):
    raise NotImplementedError("write your pallas kernel here")


# speedup vs baseline: 1.2209x; 1.2209x over previous
import functools

import jax
import jax.numpy as jnp
from jax import lax
from jax.experimental import pallas as pl
from jax.experimental.pallas import tpu as pltpu
from jax.experimental.pallas import tpu_sc as plsc

_K = 60
_N = 4096 * 8192
_NC = 2
_NS = 16
_NW = _NC * _NS
_PER_W = _N // _NW
_C = 32768
_NCH = _PER_W // _C
_L = 16


def _spline_body(x_hbm, ctab_hbm, out_hbm, ctab_v, xbuf, obuf):
    wid = lax.axis_index("s") * _NC + lax.axis_index("c")
    base = wid * _PER_W
    pltpu.sync_copy(ctab_hbm, ctab_v)

    def chunk_body(g, carry):
        off = base + g * _C
        pltpu.sync_copy(x_hbm.at[pl.ds(off, _C)], xbuf)

        def vec_body(j, c2):
            xv = xbuf[pl.ds(j * _L, _L)]
            t = jnp.minimum(jnp.maximum(xv, 0.0), 1.0) * float(_K - 1)
            idx = jnp.minimum(t, float(_K - 1) - 1e-5).astype(jnp.int32)
            alpha = t - idx.astype(jnp.float32)
            c0 = plsc.load_gather(ctab_v, [idx])
            c1 = plsc.load_gather(ctab_v, [idx + 1])
            obuf[pl.ds(j * _L, _L)] = c0 + alpha * (c1 - c0)
            return c2

        lax.fori_loop(0, _C // _L, vec_body, 0)
        pltpu.sync_copy(obuf, out_hbm.at[pl.ds(off, _C)])
        return carry

    lax.fori_loop(0, _NCH, chunk_body, 0)


_spline = functools.partial(
    pl.kernel,
    out_type=jax.ShapeDtypeStruct((_N,), jnp.float32),
    mesh=plsc.VectorSubcoreMesh(core_axis_name="c", subcore_axis_name="s"),
    scratch_types=[
        pltpu.VMEM((64,), jnp.float32),
        pltpu.VMEM((_C,), jnp.float32),
        pltpu.VMEM((_C,), jnp.float32),
    ],
    compiler_params=pltpu.CompilerParams(needs_layout_passes=False),
)(_spline_body)


@jax.jit
def kernel(x, coeffs):
    ctab = jnp.pad(coeffs, (0, 64 - _K))
    out = _spline(x.reshape(-1), ctab)
    return out.reshape(x.shape)


# speedup vs baseline: 2.3445x; 1.9204x over previous
import functools

import jax
import jax.numpy as jnp
from jax import lax
from jax.experimental import pallas as pl
from jax.experimental.pallas import tpu as pltpu
from jax.experimental.pallas import tpu_sc as plsc

_K = 60
_N = 4096 * 8192
_NC = 2
_NS = 16
_NW = _NC * _NS
_PER_W = _N // _NW
_C = 32768
_NCH = _PER_W // _C
_L = 16


def _spline_body(x_hbm, ctab_hbm, out_hbm, ctab_v, dtab_v, xbuf, obuf):
    wid = lax.axis_index("s") * _NC + lax.axis_index("c")
    base = wid * _PER_W
    pltpu.sync_copy(ctab_hbm, ctab_v.at[pl.ds(0, 64)])

    for k in range(4):
        hi = plsc.load_gather(ctab_v, [lax.iota(jnp.int32, _L) + (k * _L + 1)])
        dtab_v[pl.ds(k * _L, _L)] = hi - ctab_v[pl.ds(k * _L, _L)]

    def chunk_body(g, carry):
        off = base + g * _C
        pltpu.sync_copy(x_hbm.at[pl.ds(off, _C)], xbuf)

        @plsc.parallel_loop(0, _C // _L, unroll=8)
        def vec_body(j):
            xv = xbuf[pl.ds(j * _L, _L)]
            t = jnp.minimum(jnp.maximum(xv, 0.0), 1.0) * float(_K - 1)
            idx = jnp.minimum(t, float(_K - 1) - 1e-5).astype(jnp.int32)
            alpha = t - idx.astype(jnp.float32)
            c0 = plsc.load_gather(ctab_v, [idx])
            d = plsc.load_gather(dtab_v, [idx])
            obuf[pl.ds(j * _L, _L)] = c0 + alpha * d

        pltpu.sync_copy(obuf, out_hbm.at[pl.ds(off, _C)])
        return carry

    lax.fori_loop(0, _NCH, chunk_body, 0)


_spline = functools.partial(
    pl.kernel,
    out_type=jax.ShapeDtypeStruct((_N,), jnp.float32),
    mesh=plsc.VectorSubcoreMesh(core_axis_name="c", subcore_axis_name="s"),
    scratch_types=[
        pltpu.VMEM((80,), jnp.float32),
        pltpu.VMEM((64,), jnp.float32),
        pltpu.VMEM((_C,), jnp.float32),
        pltpu.VMEM((_C,), jnp.float32),
    ],
    compiler_params=pltpu.CompilerParams(needs_layout_passes=False),
)(_spline_body)


@jax.jit
def kernel(x, coeffs):
    ctab = jnp.pad(coeffs, (0, 64 - _K))
    out = _spline(x.reshape(-1), ctab)
    return out.reshape(x.shape)


# speedup vs baseline: 2.9465x; 1.2568x over previous
import functools

import jax
import jax.numpy as jnp
from jax import lax
from jax.experimental import pallas as pl
from jax.experimental.pallas import tpu as pltpu
from jax.experimental.pallas import tpu_sc as plsc

_K = 60
_N = 4096 * 8192
_NC = 2
_NS = 16
_NW = _NC * _NS
_PER_W = _N // _NW
_C = 16384
_NCH = _PER_W // _C
_L = 16


def _spline_body(
    x_hbm, ctab_hbm, out_hbm,
    ctab_v, dtab_v, xbuf0, xbuf1, obuf0, obuf1,
    isem0, isem1, osem0, osem1,
):
    wid = lax.axis_index("s") * _NC + lax.axis_index("c")
    base = wid * _PER_W
    pltpu.sync_copy(ctab_hbm, ctab_v.at[pl.ds(0, 64)])

    for k in range(4):
        hi = plsc.load_gather(ctab_v, [lax.iota(jnp.int32, _L) + (k * _L + 1)])
        dtab_v[pl.ds(k * _L, _L)] = hi - ctab_v[pl.ds(k * _L, _L)]

    def compute(xb, ob):
        @plsc.parallel_loop(0, _C // _L, unroll=8)
        def vec_body(j):
            xv = xb[pl.ds(j * _L, _L)]
            t = jnp.minimum(jnp.maximum(xv, 0.0), 1.0) * float(_K - 1)
            idx = jnp.minimum(t, float(_K - 1) - 1e-5).astype(jnp.int32)
            alpha = t - idx.astype(jnp.float32)
            c0 = plsc.load_gather(ctab_v, [idx])
            d = plsc.load_gather(dtab_v, [idx])
            ob[pl.ds(j * _L, _L)] = c0 + alpha * d

    bufs = ((xbuf0, obuf0, isem0, osem0), (xbuf1, obuf1, isem1, osem1))

    pltpu.async_copy(x_hbm.at[pl.ds(base, _C)], xbuf0, isem0)
    pltpu.async_copy(x_hbm.at[pl.ds(base + _C, _C)], xbuf1, isem1)

    @pl.loop(0, _NCH, step=2)
    def chunk_pair(g):
        for b, (xb, ob, isem, osem) in enumerate(bufs):
            gg = g + b
            pltpu.make_async_copy(x_hbm.at[pl.ds(base, _C)], xb, isem).wait()
            @pl.when(gg >= 2)
            def _():
                pltpu.make_async_copy(ob, out_hbm.at[pl.ds(base, _C)], osem).wait()

            compute(xb, ob)
            pltpu.async_copy(ob, out_hbm.at[pl.ds(base + gg * _C, _C)], osem)

            @pl.when(gg + 2 < _NCH)
            def _():
                pltpu.async_copy(
                    x_hbm.at[pl.ds(base + (gg + 2) * _C, _C)], xb, isem
                )

    for _, ob, _, osem in bufs:
        pltpu.make_async_copy(ob, out_hbm.at[pl.ds(base, _C)], osem).wait()


_spline = functools.partial(
    pl.kernel,
    out_type=jax.ShapeDtypeStruct((_N,), jnp.float32),
    mesh=plsc.VectorSubcoreMesh(core_axis_name="c", subcore_axis_name="s"),
    scratch_types=[
        pltpu.VMEM((80,), jnp.float32),
        pltpu.VMEM((64,), jnp.float32),
        pltpu.VMEM((_C,), jnp.float32),
        pltpu.VMEM((_C,), jnp.float32),
        pltpu.VMEM((_C,), jnp.float32),
        pltpu.VMEM((_C,), jnp.float32),
        pltpu.SemaphoreType.DMA,
        pltpu.SemaphoreType.DMA,
        pltpu.SemaphoreType.DMA,
        pltpu.SemaphoreType.DMA,
    ],
    compiler_params=pltpu.CompilerParams(needs_layout_passes=False),
)(_spline_body)


@jax.jit
def kernel(x, coeffs):
    ctab = jnp.pad(coeffs, (0, 64 - _K))
    out = _spline(x.reshape(-1), ctab)
    return out.reshape(x.shape)
